# Initial kernel scaffold; baseline (speedup 1.0000x reference)
import jax
import jax.numpy as jnp
from jax.experimental import pallas as pl


def kernel(s_j, dist, nbrs, ln_g, ln_b, Wq, Wk, Wv, Wdk, bdk, Wdv, bdv, Wd, bd):
    raise NotImplementedError("write your pallas kernel here")


# speedup vs baseline: 2.2161x; 2.2161x over previous
import functools

import jax
import jax.numpy as jnp
from jax import lax
from jax.experimental import pallas as pl
from jax.experimental.pallas import tpu as pltpu
from jax.experimental.pallas import tpu_sc as plsc

N_NODES = 10000
FEAT = 128
NUM_HEADS = 2
HF = NUM_HEADS * FEAT
N_RBF = 20
CUTOFF = 5.0
N_EDGES = 160000

_NODE_BLK = 400
_EDGE_BLK = 1600
_CH = 128
_NW = 32
_NCHUNK = N_EDGES // _CH


def _silu(x):
    return x * jax.nn.sigmoid(x)


def _node_body(s_ref, g_ref, b_ref, wq_ref, wk_ref, wv_ref, q_ref, k_ref, v_ref):
    x = s_ref[...]
    mu = jnp.mean(x, axis=1, keepdims=True)
    xc = x - mu
    var = jnp.mean(xc * xc, axis=1, keepdims=True)
    xn = xc * lax.rsqrt(var + 1e-5) * g_ref[...] + b_ref[...]
    q_ref[...] = jnp.dot(xn, wq_ref[...], preferred_element_type=jnp.float32)
    k_ref[...] = jnp.dot(xn, wk_ref[...], preferred_element_type=jnp.float32)
    v_ref[...] = jnp.dot(xn, wv_ref[...], preferred_element_type=jnp.float32)


def _edge_body(dist_ref, qi_ref, kj_ref, vj_ref, wdk_ref, bdk_ref, wdv_ref,
               bdv_ref, wd_ref, bd_ref, out_ref):
    d = dist_ref[...]
    lane = lax.broadcasted_iota(jnp.int32, (1, FEAT), 1)
    width = CUTOFF / (N_RBF - 1)
    coeff = -0.5 / (width * width)
    diff = d - lane.astype(jnp.float32) * width
    rbf = jnp.where(lane < N_RBF, jnp.exp(coeff * diff * diff), 0.0)
    dk = _silu(jnp.dot(rbf, wdk_ref[...], preferred_element_type=jnp.float32)
               + bdk_ref[...])
    dv = _silu(jnp.dot(rbf, wdv_ref[...], preferred_element_type=jnp.float32)
               + bdv_ref[...])
    t = qi_ref[...] * kj_ref[...] * dk
    a0 = _silu(jnp.sum(t[:, :FEAT], axis=1, keepdims=True))
    a1 = _silu(jnp.sum(t[:, FEAT:], axis=1, keepdims=True))
    w = vj_ref[...] * dv
    msg = jnp.concatenate([w[:, :FEAT] * a0, w[:, FEAT:] * a1], axis=1)
    out_ref[...] = (jnp.dot(msg, wd_ref[...], preferred_element_type=jnp.float32)
                    + bd_ref[...])


def _sc_gather_body(q_hbm, k_hbm, v_hbm, ii_hbm, jj_hbm, qi_hbm, kj_hbm, vj_hbm,
                    ii_v, jj_v, bq, bk, bv, sem):
    nc = 2
    wid = lax.axis_index("s") * nc + lax.axis_index("c")

    def body(t, carry):
        c = wid + t * _NW

        @pl.when(c < _NCHUNK)
        def _():
            base = c * _CH
            pltpu.sync_copy(ii_hbm.at[pl.ds(base, _CH)], ii_v)
            pltpu.sync_copy(jj_hbm.at[pl.ds(base, _CH)], jj_v)
            c1 = pltpu.async_copy(q_hbm.at[ii_v], bq, sem)
            c2 = pltpu.async_copy(k_hbm.at[jj_v], bk, sem)
            c3 = pltpu.async_copy(v_hbm.at[jj_v], bv, sem)
            c1.wait()
            c2.wait()
            c3.wait()
            pltpu.sync_copy(bq, qi_hbm.at[pl.ds(base, _CH)])
            pltpu.sync_copy(bk, kj_hbm.at[pl.ds(base, _CH)])
            pltpu.sync_copy(bv, vj_hbm.at[pl.ds(base, _CH)])

        return carry

    iters = (_NCHUNK + _NW - 1) // _NW
    lax.fori_loop(0, iters, body, None)


def _project_nodes(s_j, ln_g, ln_b, Wq, Wk, Wv):
    full = lambda shape: pl.BlockSpec(shape, lambda i: (0, 0))
    return pl.pallas_call(
        _node_body,
        grid=(N_NODES // _NODE_BLK,),
        in_specs=[
            pl.BlockSpec((_NODE_BLK, FEAT), lambda i: (i, 0)),
            full((1, FEAT)),
            full((1, FEAT)),
            full((FEAT, HF)),
            full((FEAT, HF)),
            full((FEAT, HF)),
        ],
        out_specs=[pl.BlockSpec((_NODE_BLK, HF), lambda i: (i, 0))] * 3,
        out_shape=[jax.ShapeDtypeStruct((N_NODES, HF), jnp.float32)] * 3,
    )(s_j, ln_g.reshape(1, FEAT), ln_b.reshape(1, FEAT), Wq, Wk, Wv)


def _gather_edges(q, k, v, ii, jj):
    mesh = plsc.VectorSubcoreMesh(core_axis_name="c", subcore_axis_name="s")
    call = functools.partial(
        pl.kernel,
        mesh=mesh,
        out_type=[jax.ShapeDtypeStruct((N_EDGES, HF), jnp.float32)] * 3,
        scratch_types=[
            pltpu.VMEM((_CH,), jnp.int32),
            pltpu.VMEM((_CH,), jnp.int32),
            pltpu.VMEM((_CH, HF), jnp.float32),
            pltpu.VMEM((_CH, HF), jnp.float32),
            pltpu.VMEM((_CH, HF), jnp.float32),
            pltpu.SemaphoreType.DMA,
        ],
    )(_sc_gather_body)
    return call(q, k, v, ii, jj)


def _edge_compute(dist, qi, kj, vj, Wdk_p, bdk, Wdv_p, bdv, Wd, bd):
    full = lambda shape: pl.BlockSpec(shape, lambda i: (0, 0))
    eb = pl.BlockSpec((_EDGE_BLK, HF), lambda i: (i, 0))
    return pl.pallas_call(
        _edge_body,
        grid=(N_EDGES // _EDGE_BLK,),
        in_specs=[
            pl.BlockSpec((_EDGE_BLK, 1), lambda i: (i, 0)),
            eb, eb, eb,
            full((FEAT, HF)),
            full((1, HF)),
            full((FEAT, HF)),
            full((1, HF)),
            full((HF, 3 * FEAT)),
            full((1, 3 * FEAT)),
        ],
        out_specs=pl.BlockSpec((_EDGE_BLK, 3 * FEAT), lambda i: (i, 0)),
        out_shape=jax.ShapeDtypeStruct((N_EDGES, 3 * FEAT), jnp.float32),
    )(dist.reshape(N_EDGES, 1), qi, kj, vj, Wdk_p, bdk.reshape(1, HF),
      Wdv_p, bdv.reshape(1, HF), Wd, bd.reshape(1, 3 * FEAT))


def kernel(s_j, dist, nbrs, ln_g, ln_b, Wq, Wk, Wv, Wdk, bdk, Wdv, bdv, Wd, bd):
    q, k, v = _project_nodes(s_j, ln_g, ln_b, Wq, Wk, Wv)
    ii = nbrs[:, 0].astype(jnp.int32)
    jj = nbrs[:, 1].astype(jnp.int32)
    qi, kj, vj = _gather_edges(q, k, v, ii, jj)
    Wdk_p = jnp.zeros((FEAT, HF), jnp.float32).at[:N_RBF].set(Wdk)
    Wdv_p = jnp.zeros((FEAT, HF), jnp.float32).at[:N_RBF].set(Wdv)
    out = _edge_compute(dist, qi, kj, vj, Wdk_p, bdk, Wdv_p, bdv, Wd, bd)
    return out.reshape(N_EDGES, 3, FEAT)
